# Initial kernel scaffold; baseline (speedup 1.0000x reference)
import jax
import jax.numpy as jnp
from jax.experimental import pallas as pl


def kernel(in_core_feats, aux_feats, id_map, roi_ids, pos_ids, weight, bias):
    raise NotImplementedError("write your pallas kernel here")


# speedup vs baseline: 1.5179x; 1.5179x over previous
import functools

import jax
import jax.numpy as jnp
from jax import lax
from jax.experimental import pallas as pl
from jax.experimental.pallas import tpu as pltpu
from jax.experimental.pallas import tpu_sc as plsc

NC, NS, L = 2, 16, 16
NW = NC * NS
N_PAD = 51200
NPW = N_PAD // NW
RH, RW = 64, 64
C = 128
KTAPS = 9
SG = 32
NSG = NPW // SG
BN = 1024


def _sc_gather_body(feats_hbm, idrows_hbm, roi_hbm, px_hbm, py_hbm, out_hbm,
                    roi_v, px_v, py_v, idrow_v, cids_v, stage_v,
                    sem_i, sem_f0, sem_f1, sem_w0, sem_w1):
    cid = lax.axis_index("c")
    sid = lax.axis_index("s")
    wid = sid * NC + cid
    base = wid * NPW
    pltpu.sync_copy(roi_hbm.at[pl.ds(base, NPW)], roi_v)
    pltpu.sync_copy(px_hbm.at[pl.ds(base, NPW)], px_v)
    pltpu.sync_copy(py_hbm.at[pl.ds(base, NPW)], py_v)
    lane = lax.iota(jnp.int32, L)
    pad_row = feats_hbm.shape[0] - 1
    sem_f = (sem_f0, sem_f1)
    sem_w = (sem_w0, sem_w1)

    def compute_ids(s, b):
        off = s * SG
        id_cps = []
        for g2 in range(SG // L):
            py = py_v[pl.ds(off + g2 * L, L)]
            roi = roi_v[pl.ds(off + g2 * L, L)]
            start = jnp.clip(py - 1, 0, RH - 3)
            qg = roi * (RH // 2) + (start >> 1)
            for j in range(2):
                id_cps.append(pltpu.async_copy(
                    idrows_hbm.at[qg + j],
                    idrow_v.at[pl.ds((b * 2 + j) * SG + g2 * L, L)], sem_i))
        for cp in id_cps:
            cp.wait()
        for g2 in range(SG // L):
            px = px_v[pl.ds(off + g2 * L, L)]
            py = py_v[pl.ds(off + g2 * L, L)]
            start = jnp.clip(py - 1, 0, RH - 3)
            off0 = (start & 1) * RW
            for k in range(KTAPS):
                dy = k // 3 - 1
                dx = k % 3 - 1
                yy = py + dy
                xx = px + dx
                mask = (yy < 0) | (yy >= RH) | (xx < 0) | (xx >= RW)
                lrow = jnp.clip(yy, 0, RH - 1) - start
                col = jnp.clip(xx, 0, RW - 1)
                t = off0 + lrow * RW + col
                raw = plsc.load_gather(
                    idrow_v,
                    [(b * 2 + (t >> 7)) * SG + g2 * L + lane, t & 127])
                cids_v[b * KTAPS + k, pl.ds(g2 * L, L)] = (
                    jnp.where(mask, pad_row, raw))

    def feat_cps(b, make):
        mk = pltpu.make_async_copy if make else pltpu.async_copy
        return [mk(feats_hbm.at[cids_v.at[b * KTAPS + k, pl.ds(0, SG)]],
                   stage_v.at[b, k], sem_f[b])
                for k in range(KTAPS)]

    def write_cps(s, b, make):
        mk = pltpu.make_async_copy if make else pltpu.async_copy
        off = base + s * SG
        return [mk(stage_v.at[b, k], out_hbm.at[k, pl.ds(off, SG)], sem_w[b])
                for k in range(KTAPS)]

    def pair(t, carry):
        for b in range(2):
            s = 2 * t + b
            @pl.when(t >= 1)
            def _():
                for cp in write_cps(s - 2, b, True):
                    cp.wait()
            compute_ids(s, b)
            feat_cps(b, False)
            if b == 1:
                for cp in feat_cps(0, True):
                    cp.wait()
                write_cps(s - 1, 0, False)
            else:
                @pl.when(t >= 1)
                def _():
                    for cp in feat_cps(1, True):
                        cp.wait()
                    write_cps(s - 1, 1, False)
        return carry

    lax.fori_loop(0, NSG // 2, pair, 0)
    for cp in feat_cps(1, True):
        cp.wait()
    write_cps(NSG - 1, 1, False)
    for cp in write_cps(NSG - 2, 0, True):
        cp.wait()
    for cp in write_cps(NSG - 1, 1, True):
        cp.wait()


@functools.partial(
    pl.kernel,
    out_type=jax.ShapeDtypeStruct((KTAPS, N_PAD, C), jnp.float32),
    mesh=plsc.VectorSubcoreMesh(core_axis_name="c", subcore_axis_name="s"),
    scratch_types=[
        pltpu.VMEM((NPW,), jnp.int32),
        pltpu.VMEM((NPW,), jnp.int32),
        pltpu.VMEM((NPW,), jnp.int32),
        pltpu.VMEM((4 * SG, 2 * RW), jnp.int32),
        pltpu.VMEM((2 * KTAPS, 128), jnp.int32),
        pltpu.VMEM((2, KTAPS, SG, C), jnp.float32),
        pltpu.SemaphoreType.DMA,
        pltpu.SemaphoreType.DMA,
        pltpu.SemaphoreType.DMA,
        pltpu.SemaphoreType.DMA,
        pltpu.SemaphoreType.DMA,
    ],
    compiler_params=pltpu.CompilerParams(needs_layout_passes=False),
)
def _sc_gather(*args):
    _sc_gather_body(*args)


def _tc_matmul_body(g_ref, w_ref, b_ref, o_ref):
    acc = jnp.zeros((BN, C), jnp.float32)
    for k in range(KTAPS):
        acc += jnp.dot(g_ref[k], w_ref[k], preferred_element_type=jnp.float32)
    o_ref[...] = acc + b_ref[...]


def _tc_matmul(gathered, w_blocks, bias2d):
    return pl.pallas_call(
        _tc_matmul_body,
        grid=(N_PAD // BN,),
        in_specs=[
            pl.BlockSpec((KTAPS, BN, C), lambda i: (0, i, 0)),
            pl.BlockSpec((KTAPS, C, C), lambda i: (0, 0, 0)),
            pl.BlockSpec((1, C), lambda i: (0, 0)),
        ],
        out_specs=pl.BlockSpec((BN, C), lambda i: (i, 0)),
        out_shape=jax.ShapeDtypeStruct((N_PAD, C), jnp.float32),
    )(gathered, w_blocks, bias2d)


@jax.jit
def kernel(in_core_feats, aux_feats, id_map, roi_ids, pos_ids, weight, bias):
    n = in_core_feats.shape[0]
    all_feats = jnp.concatenate(
        [in_core_feats, aux_feats,
         jnp.zeros((1, C), in_core_feats.dtype)], axis=0)
    idrows = id_map.reshape(-1, 2 * RW)
    pad_n = N_PAD - n
    roi = jnp.pad(roi_ids, (0, pad_n))
    px = jnp.pad(pos_ids[:, 0], (0, pad_n))
    py = jnp.pad(pos_ids[:, 1], (0, pad_n))
    gathered = _sc_gather(all_feats, idrows, roi, px, py)
    w_blocks = weight.T.reshape(KTAPS, C, C)
    out = _tc_matmul(gathered, w_blocks, bias.reshape(1, C))
    return out[:n]


# speedup vs baseline: 1.9617x; 1.2924x over previous
import functools

import jax
import jax.numpy as jnp
from jax import lax
from jax.experimental import pallas as pl
from jax.experimental.pallas import tpu as pltpu
from jax.experimental.pallas import tpu_sc as plsc

NC, NS, L = 2, 16, 16
NW = NC * NS
N_PAD = 52224
NPW = N_PAD // NW
RH, RW = 64, 64
C = 128
KTAPS = 9
SG = 48
NSG = NPW // SG
VPAD = 60416
BV = 1024


def _sc_body(p_hbm, idrows_hbm, roi_hbm, px_hbm, py_hbm, out_hbm,
             roi_v, px_v, py_v, idrow_v, cids_v, acc_v,
             sem_i, sem_g0, sem_g1, sem_o0, sem_o1):
    cid = lax.axis_index("c")
    sid = lax.axis_index("s")
    wid = sid * NC + cid
    base = wid * NPW
    pltpu.sync_copy(roi_hbm.at[pl.ds(base, NPW)], roi_v)
    pltpu.sync_copy(px_hbm.at[pl.ds(base, NPW)], px_v)
    pltpu.sync_copy(py_hbm.at[pl.ds(base, NPW)], py_v)
    lane = lax.iota(jnp.int32, L)
    pad_row = VPAD - 416
    zeros = jnp.zeros((L,), jnp.float32)
    sem_g = (sem_g0, sem_g1)
    sem_o = (sem_o0, sem_o1)
    NG = SG // L

    def compute_ids(s, b):
        off = s * SG
        id_cps = []
        for g2 in range(NG):
            py = py_v[pl.ds(off + g2 * L, L)]
            roi = roi_v[pl.ds(off + g2 * L, L)]
            start = jnp.clip(py - 1, 0, RH - 3)
            qg = roi * (RH // 2) + (start >> 1)
            for j in range(2):
                id_cps.append(pltpu.async_copy(
                    idrows_hbm.at[qg + j],
                    idrow_v.at[pl.ds(((b * 2 + j) * NG + g2) * L, L)], sem_i))
        for cp in id_cps:
            cp.wait()
        for g2 in range(NG):
            px = px_v[pl.ds(off + g2 * L, L)]
            py = py_v[pl.ds(off + g2 * L, L)]
            start = jnp.clip(py - 1, 0, RH - 3)
            off0 = (start & 1) * RW
            for k in range(KTAPS):
                dy = k // 3 - 1
                dx = k % 3 - 1
                yy = py + dy
                xx = px + dx
                mask = (yy < 0) | (yy >= RH) | (xx < 0) | (xx >= RW)
                lrow = jnp.clip(yy, 0, RH - 1) - start
                col = jnp.clip(xx, 0, RW - 1)
                t = off0 + lrow * RW + col
                raw = plsc.load_gather(
                    idrow_v,
                    [((b * 2 + (t >> 7)) * NG + g2) * L + lane, t & 127])
                cids_v[b * KTAPS + k, pl.ds(g2 * L, L)] = (
                    jnp.where(mask, pad_row, raw) + k * VPAD)

    def zero_acc(b):
        for r in range(SG):
            for c8 in range(C // L):
                acc_v[b * SG + r, pl.ds(c8 * L, L)] = zeros

    def gadd_cps(b, make):
        mk = pltpu.make_async_copy if make else pltpu.async_copy
        if make:
            return [mk(p_hbm.at[cids_v.at[b * KTAPS + k, pl.ds(0, SG)]],
                       acc_v.at[pl.ds(b * SG, SG)], sem_g[b])
                    for k in range(KTAPS)]
        return [pltpu.async_copy(
            p_hbm.at[cids_v.at[b * KTAPS + k, pl.ds(0, SG)]],
            acc_v.at[pl.ds(b * SG, SG)], sem_g[b], add=True)
            for k in range(KTAPS)]

    def out_cp(s, b, make):
        mk = pltpu.make_async_copy if make else pltpu.async_copy
        return mk(acc_v.at[pl.ds(b * SG, SG)],
                  out_hbm.at[pl.ds(base + s * SG, SG)], sem_o[b])

    def pair(t, carry):
        for b in range(2):
            s = 2 * t + b
            @pl.when(t >= 1)
            def _():
                out_cp(s - 2, b, True).wait()
            compute_ids(s, b)
            zero_acc(b)
            gadd_cps(b, False)
            if b == 1:
                for cp in gadd_cps(0, True):
                    cp.wait()
                out_cp(s - 1, 0, False)
            else:
                @pl.when(t >= 1)
                def _():
                    for cp in gadd_cps(1, True):
                        cp.wait()
                    out_cp(s - 1, 1, False)
        return carry

    lax.fori_loop(0, NSG // 2, pair, 0)
    for cp in gadd_cps(1, True):
        cp.wait()
    out_cp(NSG - 1, 1, False)
    out_cp(NSG - 2, 0, True).wait()
    out_cp(NSG - 1, 1, True).wait()


@functools.partial(
    pl.kernel,
    out_type=jax.ShapeDtypeStruct((N_PAD, C), jnp.float32),
    mesh=plsc.VectorSubcoreMesh(core_axis_name="c", subcore_axis_name="s"),
    scratch_types=[
        pltpu.VMEM((NPW,), jnp.int32),
        pltpu.VMEM((NPW,), jnp.int32),
        pltpu.VMEM((NPW,), jnp.int32),
        pltpu.VMEM((4 * SG, 2 * RW), jnp.int32),
        pltpu.VMEM((2 * KTAPS, 128), jnp.int32),
        pltpu.VMEM((2 * SG, C), jnp.float32),
        pltpu.SemaphoreType.DMA,
        pltpu.SemaphoreType.DMA,
        pltpu.SemaphoreType.DMA,
        pltpu.SemaphoreType.DMA,
        pltpu.SemaphoreType.DMA,
    ],
    compiler_params=pltpu.CompilerParams(needs_layout_passes=False),
)
def _sc_gather_add(*args):
    _sc_body(*args)


def _tc_project_body(f_ref, w_ref, b_ref, o_ref):
    o_ref[0] = (jnp.dot(f_ref[...], w_ref[0],
                        preferred_element_type=jnp.float32) + b_ref[...])


def _tc_project(feats_pad, w_blocks, bias9):
    return pl.pallas_call(
        _tc_project_body,
        grid=(KTAPS, VPAD // BV),
        in_specs=[
            pl.BlockSpec((BV, C), lambda k, i: (i, 0)),
            pl.BlockSpec((1, C, C), lambda k, i: (k, 0, 0)),
            pl.BlockSpec((1, C), lambda k, i: (0, 0)),
        ],
        out_specs=pl.BlockSpec((1, BV, C), lambda k, i: (k, i, 0)),
        out_shape=jax.ShapeDtypeStruct((KTAPS, VPAD, C), jnp.float32),
    )(feats_pad, w_blocks, bias9)


@jax.jit
def kernel(in_core_feats, aux_feats, id_map, roi_ids, pos_ids, weight, bias):
    n = in_core_feats.shape[0]
    a = aux_feats.shape[0]
    feats_pad = jnp.zeros((VPAD, C), jnp.float32)
    feats_pad = lax.dynamic_update_slice(feats_pad, in_core_feats, (0, 0))
    feats_pad = lax.dynamic_update_slice(feats_pad, aux_feats, (n, 0))
    w_blocks = weight.T.reshape(KTAPS, C, C)
    p = _tc_project(feats_pad, w_blocks, (bias / KTAPS).reshape(1, C))
    p2d = p.reshape(KTAPS * VPAD, C)
    idrows = id_map.reshape(-1, 2 * RW)
    pad_n = N_PAD - n
    roi = jnp.pad(roi_ids, (0, pad_n))
    px = jnp.pad(pos_ids[:, 0], (0, pad_n))
    py = jnp.pad(pos_ids[:, 1], (0, pad_n))
    out = _sc_gather_add(p2d, idrows, roi, px, py)
    return out[:n]


# speedup vs baseline: 1.9797x; 1.0092x over previous
import functools

import jax
import jax.numpy as jnp
from jax import lax
from jax.experimental import pallas as pl
from jax.experimental.pallas import tpu as pltpu
from jax.experimental.pallas import tpu_sc as plsc

NC, NS, L = 2, 16, 16
NW = NC * NS
N_PAD = 52224
NPW = N_PAD // NW
RH, RW = 64, 64
C = 128
KTAPS = 9
SG = 48
NSG = NPW // SG
VPAD = 60416
BV = 1024


def _sc_body(p_hbm, idrows_hbm, roi_hbm, px_hbm, py_hbm, out_hbm,
             roi_v, px_v, py_v, idrow_v, cids_v, acc_v,
             sem_i0, sem_i1, sem_g0, sem_g1, sem_o0, sem_o1):
    cid = lax.axis_index("c")
    sid = lax.axis_index("s")
    wid = sid * NC + cid
    base = wid * NPW
    pltpu.sync_copy(roi_hbm.at[pl.ds(base, NPW)], roi_v)
    pltpu.sync_copy(px_hbm.at[pl.ds(base, NPW)], px_v)
    pltpu.sync_copy(py_hbm.at[pl.ds(base, NPW)], py_v)
    lane = lax.iota(jnp.int32, L)
    pad_row = VPAD - 416
    zeros = jnp.zeros((L,), jnp.float32)
    sem_i = (sem_i0, sem_i1)
    sem_g = (sem_g0, sem_g1)
    sem_o = (sem_o0, sem_o1)
    NG = SG // L

    def id_cps(s, b, make):
        mk = pltpu.make_async_copy if make else pltpu.async_copy
        off = s * SG
        cps = []
        for g2 in range(NG):
            py = py_v[pl.ds(off + g2 * L, L)]
            roi = roi_v[pl.ds(off + g2 * L, L)]
            start = jnp.clip(py - 1, 0, RH - 3)
            qg = roi * (RH // 2) + (start >> 1)
            for j in range(2):
                cps.append(mk(
                    idrows_hbm.at[qg + j],
                    idrow_v.at[pl.ds(((b * 2 + j) * NG + g2) * L, L)],
                    sem_i[b]))
        return cps

    def compute_ids(s, b):
        off = s * SG
        for g2 in range(NG):
            px = px_v[pl.ds(off + g2 * L, L)]
            py = py_v[pl.ds(off + g2 * L, L)]
            start = jnp.clip(py - 1, 0, RH - 3)
            off0 = (start & 1) * RW
            for k in range(KTAPS):
                dy = k // 3 - 1
                dx = k % 3 - 1
                yy = py + dy
                xx = px + dx
                mask = (yy < 0) | (yy >= RH) | (xx < 0) | (xx >= RW)
                lrow = jnp.clip(yy, 0, RH - 1) - start
                col = jnp.clip(xx, 0, RW - 1)
                t = off0 + lrow * RW + col
                raw = plsc.load_gather(
                    idrow_v,
                    [((b * 2 + (t >> 7)) * NG + g2) * L + lane, t & 127])
                cids_v[b * KTAPS + k, pl.ds(g2 * L, L)] = (
                    jnp.where(mask, pad_row, raw) + k * VPAD)

    def zero_acc(b):
        for r in range(SG):
            for c8 in range(C // L):
                acc_v[b * SG + r, pl.ds(c8 * L, L)] = zeros

    def gadd_cps(b, make):
        mk = pltpu.make_async_copy if make else pltpu.async_copy
        if make:
            return [mk(p_hbm.at[cids_v.at[b * KTAPS + k, pl.ds(0, SG)]],
                       acc_v.at[pl.ds(b * SG, SG)], sem_g[b])
                    for k in range(KTAPS)]
        return [pltpu.async_copy(
            p_hbm.at[cids_v.at[b * KTAPS + k, pl.ds(0, SG)]],
            acc_v.at[pl.ds(b * SG, SG)], sem_g[b], add=True)
            for k in range(KTAPS)]

    def out_cp(s, b, make):
        mk = pltpu.make_async_copy if make else pltpu.async_copy
        return mk(acc_v.at[pl.ds(b * SG, SG)],
                  out_hbm.at[pl.ds(base + s * SG, SG)], sem_o[b])

    def pair(t, carry):
        for b in range(2):
            s = 2 * t + b
            for cp in id_cps(s, b, True):
                cp.wait()
            id_cps(jnp.minimum(s + 1, NSG - 1), 1 - b, False)
            @pl.when(t >= 1)
            def _():
                out_cp(s - 2, b, True).wait()
            compute_ids(s, b)
            zero_acc(b)
            gadd_cps(b, False)
            if b == 1:
                for cp in gadd_cps(0, True):
                    cp.wait()
                out_cp(s - 1, 0, False)
            else:
                @pl.when(t >= 1)
                def _():
                    for cp in gadd_cps(1, True):
                        cp.wait()
                    out_cp(s - 1, 1, False)
        return carry

    id_cps(0, 0, False)
    lax.fori_loop(0, NSG // 2, pair, 0)
    for cp in gadd_cps(1, True):
        cp.wait()
    out_cp(NSG - 1, 1, False)
    for cp in id_cps(NSG - 1, 0, True):
        cp.wait()
    out_cp(NSG - 2, 0, True).wait()
    out_cp(NSG - 1, 1, True).wait()


@functools.partial(
    pl.kernel,
    out_type=jax.ShapeDtypeStruct((N_PAD, C), jnp.float32),
    mesh=plsc.VectorSubcoreMesh(core_axis_name="c", subcore_axis_name="s"),
    scratch_types=[
        pltpu.VMEM((NPW,), jnp.int32),
        pltpu.VMEM((NPW,), jnp.int32),
        pltpu.VMEM((NPW,), jnp.int32),
        pltpu.VMEM((4 * SG, 2 * RW), jnp.int32),
        pltpu.VMEM((2 * KTAPS, 128), jnp.int32),
        pltpu.VMEM((2 * SG, C), jnp.float32),
        pltpu.SemaphoreType.DMA,
        pltpu.SemaphoreType.DMA,
        pltpu.SemaphoreType.DMA,
        pltpu.SemaphoreType.DMA,
        pltpu.SemaphoreType.DMA,
        pltpu.SemaphoreType.DMA,
    ],
    compiler_params=pltpu.CompilerParams(needs_layout_passes=False),
)
def _sc_gather_add(*args):
    _sc_body(*args)


def _tc_project_body(f_ref, w_ref, b_ref, o_ref):
    o_ref[0] = (jnp.dot(f_ref[...], w_ref[0],
                        preferred_element_type=jnp.float32) + b_ref[...])


def _tc_project(feats_pad, w_blocks, bias9):
    return pl.pallas_call(
        _tc_project_body,
        grid=(KTAPS, VPAD // BV),
        in_specs=[
            pl.BlockSpec((BV, C), lambda k, i: (i, 0)),
            pl.BlockSpec((1, C, C), lambda k, i: (k, 0, 0)),
            pl.BlockSpec((1, C), lambda k, i: (0, 0)),
        ],
        out_specs=pl.BlockSpec((1, BV, C), lambda k, i: (k, i, 0)),
        out_shape=jax.ShapeDtypeStruct((KTAPS, VPAD, C), jnp.float32),
    )(feats_pad, w_blocks, bias9)


@jax.jit
def kernel(in_core_feats, aux_feats, id_map, roi_ids, pos_ids, weight, bias):
    n = in_core_feats.shape[0]
    a = aux_feats.shape[0]
    feats_pad = jnp.zeros((VPAD, C), jnp.float32)
    feats_pad = lax.dynamic_update_slice(feats_pad, in_core_feats, (0, 0))
    feats_pad = lax.dynamic_update_slice(feats_pad, aux_feats, (n, 0))
    w_blocks = weight.T.reshape(KTAPS, C, C)
    p = _tc_project(feats_pad, w_blocks, (bias / KTAPS).reshape(1, C))
    p2d = p.reshape(KTAPS * VPAD, C)
    idrows = id_map.reshape(-1, 2 * RW)
    pad_n = N_PAD - n
    roi = jnp.pad(roi_ids, (0, pad_n))
    px = jnp.pad(pos_ids[:, 0], (0, pad_n))
    py = jnp.pad(pos_ids[:, 1], (0, pad_n))
    out = _sc_gather_add(p2d, idrows, roi, px, py)
    return out[:n]


# speedup vs baseline: 1.9902x; 1.0053x over previous
import functools

import jax
import jax.numpy as jnp
from jax import lax
from jax.experimental import pallas as pl
from jax.experimental.pallas import tpu as pltpu
from jax.experimental.pallas import tpu_sc as plsc

NC, NS, L = 2, 16, 16
NW = NC * NS
N_PAD = 52224
NPW = N_PAD // NW
RH, RW = 64, 64
C = 128
KTAPS = 9
SG = 48
NSG = NPW // SG
VPAD = 60416
BV = 1024


def _sc_body(p_hbm, idrows_hbm, roi_hbm, px_hbm, py_hbm, out_hbm,
             roi_v, px_v, py_v, idrow_v, cids_v, acc_v,
             sem_i0, sem_i1, sem_g0, sem_g1, sem_o0, sem_o1):
    cid = lax.axis_index("c")
    sid = lax.axis_index("s")
    wid = sid * NC + cid
    base = wid * NPW
    pltpu.sync_copy(roi_hbm.at[pl.ds(base, NPW)], roi_v)
    pltpu.sync_copy(px_hbm.at[pl.ds(base, NPW)], px_v)
    pltpu.sync_copy(py_hbm.at[pl.ds(base, NPW)], py_v)
    lane = lax.iota(jnp.int32, L)
    pad_row = VPAD - 416
    zeros = jnp.zeros((L,), jnp.float32)
    sem_i = (sem_i0, sem_i1)
    sem_g = (sem_g0, sem_g1)
    sem_o = (sem_o0, sem_o1)
    NG = SG // L

    def id_cps(s, b, make):
        mk = pltpu.make_async_copy if make else pltpu.async_copy
        off = s * SG
        cps = []
        for g2 in range(NG):
            py = py_v[pl.ds(off + g2 * L, L)]
            roi = roi_v[pl.ds(off + g2 * L, L)]
            start = jnp.clip(py - 1, 0, RH - 3)
            qg = roi * (RH // 2) + (start >> 1)
            for j in range(2):
                cps.append(mk(
                    idrows_hbm.at[qg + j],
                    idrow_v.at[pl.ds(((b * 2 + j) * NG + g2) * L, L)],
                    sem_i[b]))
        return cps

    def compute_ids(s, b):
        off = s * SG
        for g2 in range(NG):
            px = px_v[pl.ds(off + g2 * L, L)]
            py = py_v[pl.ds(off + g2 * L, L)]
            start = jnp.clip(py - 1, 0, RH - 3)
            off0 = (start & 1) * RW
            for k in range(KTAPS):
                dy = k // 3 - 1
                dx = k % 3 - 1
                yy = py + dy
                xx = px + dx
                mask = (yy < 0) | (yy >= RH) | (xx < 0) | (xx >= RW)
                lrow = jnp.clip(yy, 0, RH - 1) - start
                col = jnp.clip(xx, 0, RW - 1)
                t = off0 + lrow * RW + col
                raw = plsc.load_gather(
                    idrow_v,
                    [((b * 2 + (t >> 7)) * NG + g2) * L + lane, t & 127])
                cids_v[b * KTAPS + k, pl.ds(g2 * L, L)] = (
                    jnp.where(mask, pad_row, raw) + k * VPAD)

    def zero_acc(b):
        def zr(r, carry):
            for c8 in range(C // L):
                acc_v[b * 3 * SG + r, pl.ds(c8 * L, L)] = zeros
            return carry
        lax.fori_loop(0, 3 * SG, zr, 0)

    def reduce_acc(b):
        def rr(r, carry):
            for c8 in range(C // L):
                sl = pl.ds(c8 * L, L)
                acc_v[(b * 3 + 0) * SG + r, sl] = (
                    acc_v[(b * 3 + 0) * SG + r, sl]
                    + acc_v[(b * 3 + 1) * SG + r, sl]
                    + acc_v[(b * 3 + 2) * SG + r, sl])
            return carry
        lax.fori_loop(0, SG, rr, 0)

    def gadd_cps(b, make):
        mk = pltpu.make_async_copy if make else pltpu.async_copy
        if make:
            return [mk(p_hbm.at[cids_v.at[b * KTAPS + k, pl.ds(0, SG)]],
                       acc_v.at[pl.ds((b * 3 + k % 3) * SG, SG)], sem_g[b])
                    for k in range(KTAPS)]
        return [pltpu.async_copy(
            p_hbm.at[cids_v.at[b * KTAPS + k, pl.ds(0, SG)]],
            acc_v.at[pl.ds((b * 3 + k % 3) * SG, SG)], sem_g[b], add=True)
            for k in range(KTAPS)]

    def out_cp(s, b, make):
        mk = pltpu.make_async_copy if make else pltpu.async_copy
        return mk(acc_v.at[pl.ds(b * 3 * SG, SG)],
                  out_hbm.at[pl.ds(base + s * SG, SG)], sem_o[b])

    def pair(t, carry):
        for b in range(2):
            s = 2 * t + b
            for cp in id_cps(s, b, True):
                cp.wait()
            id_cps(jnp.minimum(s + 1, NSG - 1), 1 - b, False)
            @pl.when(t >= 1)
            def _():
                out_cp(s - 2, b, True).wait()
            compute_ids(s, b)
            zero_acc(b)
            gadd_cps(b, False)
            if b == 1:
                for cp in gadd_cps(0, True):
                    cp.wait()
                reduce_acc(0)
                out_cp(s - 1, 0, False)
            else:
                @pl.when(t >= 1)
                def _():
                    for cp in gadd_cps(1, True):
                        cp.wait()
                    reduce_acc(1)
                    out_cp(s - 1, 1, False)
        return carry

    id_cps(0, 0, False)
    lax.fori_loop(0, NSG // 2, pair, 0)
    for cp in gadd_cps(1, True):
        cp.wait()
    reduce_acc(1)
    out_cp(NSG - 1, 1, False)
    for cp in id_cps(NSG - 1, 0, True):
        cp.wait()
    out_cp(NSG - 2, 0, True).wait()
    out_cp(NSG - 1, 1, True).wait()


@functools.partial(
    pl.kernel,
    out_type=jax.ShapeDtypeStruct((N_PAD, C), jnp.float32),
    mesh=plsc.VectorSubcoreMesh(core_axis_name="c", subcore_axis_name="s"),
    scratch_types=[
        pltpu.VMEM((NPW,), jnp.int32),
        pltpu.VMEM((NPW,), jnp.int32),
        pltpu.VMEM((NPW,), jnp.int32),
        pltpu.VMEM((4 * SG, 2 * RW), jnp.int32),
        pltpu.VMEM((2 * KTAPS, 128), jnp.int32),
        pltpu.VMEM((6 * SG, C), jnp.float32),
        pltpu.SemaphoreType.DMA,
        pltpu.SemaphoreType.DMA,
        pltpu.SemaphoreType.DMA,
        pltpu.SemaphoreType.DMA,
        pltpu.SemaphoreType.DMA,
        pltpu.SemaphoreType.DMA,
    ],
    compiler_params=pltpu.CompilerParams(needs_layout_passes=False),
)
def _sc_gather_add(*args):
    _sc_body(*args)


def _tc_project_body(f_ref, w_ref, b_ref, o_ref):
    o_ref[0] = (jnp.dot(f_ref[...], w_ref[0],
                        preferred_element_type=jnp.float32) + b_ref[...])


def _tc_project(feats_pad, w_blocks, bias9):
    return pl.pallas_call(
        _tc_project_body,
        grid=(KTAPS, VPAD // BV),
        in_specs=[
            pl.BlockSpec((BV, C), lambda k, i: (i, 0)),
            pl.BlockSpec((1, C, C), lambda k, i: (k, 0, 0)),
            pl.BlockSpec((1, C), lambda k, i: (0, 0)),
        ],
        out_specs=pl.BlockSpec((1, BV, C), lambda k, i: (k, i, 0)),
        out_shape=jax.ShapeDtypeStruct((KTAPS, VPAD, C), jnp.float32),
    )(feats_pad, w_blocks, bias9)


@jax.jit
def kernel(in_core_feats, aux_feats, id_map, roi_ids, pos_ids, weight, bias):
    n = in_core_feats.shape[0]
    a = aux_feats.shape[0]
    feats_pad = jnp.zeros((VPAD, C), jnp.float32)
    feats_pad = lax.dynamic_update_slice(feats_pad, in_core_feats, (0, 0))
    feats_pad = lax.dynamic_update_slice(feats_pad, aux_feats, (n, 0))
    w_blocks = weight.T.reshape(KTAPS, C, C)
    p = _tc_project(feats_pad, w_blocks, (bias / KTAPS).reshape(1, C))
    p2d = p.reshape(KTAPS * VPAD, C)
    idrows = id_map.reshape(-1, 2 * RW)
    pad_n = N_PAD - n
    roi = jnp.pad(roi_ids, (0, pad_n))
    px = jnp.pad(pos_ids[:, 0], (0, pad_n))
    py = jnp.pad(pos_ids[:, 1], (0, pad_n))
    out = _sc_gather_add(p2d, idrows, roi, px, py)
    return out[:n]


# speedup vs baseline: 2.1519x; 1.0812x over previous
import functools

import jax
import jax.numpy as jnp
from jax import lax
from jax.experimental import pallas as pl
from jax.experimental.pallas import tpu as pltpu
from jax.experimental.pallas import tpu_sc as plsc

NC, NS, L = 2, 16, 16
NW = NC * NS
N_PAD = 51200
NPW = N_PAD // NW
RH, RW = 64, 64
C = 128
KTAPS = 9
SG = 80
NSG = NPW // SG
NG = SG // L
VPAD = 60416
BV = 1024


def _sc_body(p_hbm, idrows_hbm, roi_hbm, px_hbm, py_hbm, out_hbm,
             roi_v, px_v, py_v, qidx_v, idrow_v, cids_v, acc_v,
             sem_i0, sem_i1, sem_g0, sem_g1, sem_o0, sem_o1):
    sid = lax.axis_index("s")
    wid = sid * NC + lax.axis_index("c")
    base = wid * NPW
    pltpu.sync_copy(roi_hbm.at[pl.ds(base, NPW)], roi_v)
    pltpu.sync_copy(px_hbm.at[pl.ds(base, NPW)], px_v)
    pltpu.sync_copy(py_hbm.at[pl.ds(base, NPW)], py_v)
    lane = lax.iota(jnp.int32, L)
    pad_row = VPAD - 416
    zeros = jnp.zeros((L,), jnp.float32)
    sem_i = (sem_i0, sem_i1)
    sem_g = (sem_g0, sem_g1)
    sem_o = (sem_o0, sem_o1)

    def id_cps(s, b, make):
        if not make:
            off = s * SG
            for g2 in range(NG):
                py = py_v[pl.ds(off + g2 * L, L)]
                roi = roi_v[pl.ds(off + g2 * L, L)]
                start = jnp.clip(py - 1, 0, RH - 3)
                qg = roi * (RH // 2) + (start >> 1)
                qidx_v[b * 2 + 0, pl.ds(g2 * L, L)] = qg
                qidx_v[b * 2 + 1, pl.ds(g2 * L, L)] = qg + 1
        mk = pltpu.make_async_copy if make else pltpu.async_copy
        return [mk(idrows_hbm.at[qidx_v.at[b * 2 + j, pl.ds(0, SG)]],
                   idrow_v.at[pl.ds((b * 2 + j) * SG, SG)], sem_i[b])
                for j in range(2)]

    def compute_ids(s, b):
        off = s * SG
        for g2 in range(NG):
            px = px_v[pl.ds(off + g2 * L, L)]
            py = py_v[pl.ds(off + g2 * L, L)]
            start = jnp.clip(py - 1, 0, RH - 3)
            off0 = (start & 1) * RW
            for k in range(KTAPS):
                dy = k // 3 - 1
                dx = k % 3 - 1
                yy = py + dy
                xx = px + dx
                mask = (yy < 0) | (yy >= RH) | (xx < 0) | (xx >= RW)
                lrow = jnp.clip(yy, 0, RH - 1) - start
                col = jnp.clip(xx, 0, RW - 1)
                t = off0 + lrow * RW + col
                raw = plsc.load_gather(
                    idrow_v,
                    [(b * 2 + (t >> 7)) * SG + g2 * L + lane, t & 127])
                cids_v[b * KTAPS + k, pl.ds(g2 * L, L)] = (
                    jnp.where(mask, pad_row, raw) + k * VPAD)

    def zero_acc(b):
        def zr(r, carry):
            for c8 in range(C // L):
                acc_v[b * SG + r, pl.ds(c8 * L, L)] = zeros
            return carry
        lax.fori_loop(0, SG, zr, 0)

    def gadd_cps(b, make):
        if make:
            return [pltpu.make_async_copy(
                p_hbm.at[cids_v.at[b * KTAPS + k, pl.ds(0, SG)]],
                acc_v.at[pl.ds(b * SG, SG)], sem_g[b])
                for k in range(KTAPS)]
        return [pltpu.async_copy(
            p_hbm.at[cids_v.at[b * KTAPS + k, pl.ds(0, SG)]],
            acc_v.at[pl.ds(b * SG, SG)], sem_g[b], add=True)
            for k in range(KTAPS)]

    def out_cp(s, b, make):
        mk = pltpu.make_async_copy if make else pltpu.async_copy
        return mk(acc_v.at[pl.ds(b * SG, SG)],
                  out_hbm.at[pl.ds(base + s * SG, SG)], sem_o[b])

    def pair(t, carry):
        for b in range(2):
            s = 2 * t + b
            for cp in id_cps(s, b, True):
                cp.wait()
            id_cps(jnp.minimum(s + 1, NSG - 1), 1 - b, False)
            @pl.when(t >= 1)
            def _():
                out_cp(s - 2, b, True).wait()
            compute_ids(s, b)
            zero_acc(b)
            gadd_cps(b, False)
            if b == 1:
                for cp in gadd_cps(0, True):
                    cp.wait()
                out_cp(s - 1, 0, False)
            else:
                @pl.when(t >= 1)
                def _():
                    for cp in gadd_cps(1, True):
                        cp.wait()
                    out_cp(s - 1, 1, False)
        return carry

    id_cps(0, 0, False)
    lax.fori_loop(0, NSG // 2, pair, 0)
    for cp in gadd_cps(1, True):
        cp.wait()
    out_cp(NSG - 1, 1, False)
    for cp in id_cps(NSG - 1, 0, True):
        cp.wait()
    out_cp(NSG - 2, 0, True).wait()
    out_cp(NSG - 1, 1, True).wait()


@functools.partial(
    pl.kernel,
    out_type=jax.ShapeDtypeStruct((N_PAD, C), jnp.float32),
    mesh=plsc.VectorSubcoreMesh(core_axis_name="c", subcore_axis_name="s"),
    scratch_types=[
        pltpu.VMEM((NPW,), jnp.int32),
        pltpu.VMEM((NPW,), jnp.int32),
        pltpu.VMEM((NPW,), jnp.int32),
        pltpu.VMEM((4, 128), jnp.int32),
        pltpu.VMEM((4 * SG, 2 * RW), jnp.int32),
        pltpu.VMEM((2 * KTAPS, 128), jnp.int32),
        pltpu.VMEM((2 * SG, C), jnp.float32),
        pltpu.SemaphoreType.DMA,
        pltpu.SemaphoreType.DMA,
        pltpu.SemaphoreType.DMA,
        pltpu.SemaphoreType.DMA,
        pltpu.SemaphoreType.DMA,
        pltpu.SemaphoreType.DMA,
    ],
    compiler_params=pltpu.CompilerParams(needs_layout_passes=False),
)
def _sc_gather_add(*args):
    _sc_body(*args)


def _tc_project_body(f_ref, w_ref, b_ref, o_ref):
    o_ref[0] = (jnp.dot(f_ref[...], w_ref[0],
                        preferred_element_type=jnp.float32) + b_ref[...])


def _tc_project(feats_pad, w_blocks, bias9):
    return pl.pallas_call(
        _tc_project_body,
        grid=(KTAPS, VPAD // BV),
        in_specs=[
            pl.BlockSpec((BV, C), lambda k, i: (i, 0)),
            pl.BlockSpec((1, C, C), lambda k, i: (k, 0, 0)),
            pl.BlockSpec((1, C), lambda k, i: (0, 0)),
        ],
        out_specs=pl.BlockSpec((1, BV, C), lambda k, i: (k, i, 0)),
        out_shape=jax.ShapeDtypeStruct((KTAPS, VPAD, C), jnp.float32),
    )(feats_pad, w_blocks, bias9)


@jax.jit
def kernel(in_core_feats, aux_feats, id_map, roi_ids, pos_ids, weight, bias):
    n = in_core_feats.shape[0]
    feats_pad = jnp.zeros((VPAD, C), jnp.float32)
    feats_pad = lax.dynamic_update_slice(feats_pad, in_core_feats, (0, 0))
    feats_pad = lax.dynamic_update_slice(feats_pad, aux_feats, (n, 0))
    w_blocks = weight.T.reshape(KTAPS, C, C)
    p = _tc_project(feats_pad, w_blocks, (bias / KTAPS).reshape(1, C))
    p2d = p.reshape(KTAPS * VPAD, C)
    idrows = id_map.reshape(-1, 2 * RW)
    pad_n = N_PAD - n
    roi = jnp.pad(roi_ids, (0, pad_n))
    px = jnp.pad(pos_ids[:, 0], (0, pad_n))
    py = jnp.pad(pos_ids[:, 1], (0, pad_n))
    out = _sc_gather_add(p2d, idrows, roi, px, py)
    return out[:n]
